# baseline (speedup 1.0000x reference)
import functools

import jax
import jax.numpy as jnp
from jax import lax
from jax.experimental import pallas as pl
from jax.experimental.pallas import tpu as pltpu
from jax.experimental.pallas import tpu_sc as plsc

N_PAD = 10240
K = 16
DIM = 256
NW = 32
E_TOT = N_PAD * K


def _sc_gather(table, idx, chunk, tc_tiling=True, c0=None):
    V, D = table.shape
    E = idx.shape[0]
    n_tot = E // chunk
    per_sub = n_tot // 16
    if c0 is None:
        c0 = per_sub // 2
    c1 = per_sub - c0
    assert c0 % 2 == 0 and c1 % 2 == 0 and c0 > 0 and c1 > 0
    mesh = plsc.VectorSubcoreMesh(core_axis_name="c", subcore_axis_name="s")

    @functools.partial(
        pl.kernel,
        mesh=mesh,
        compiler_params=pltpu.CompilerParams(use_tc_tiling_on_sc=tc_tiling),
        out_type=jax.ShapeDtypeStruct((E, D), table.dtype),
        scratch_types=[
            pltpu.VMEM((chunk,), jnp.int32),
            pltpu.VMEM((chunk,), jnp.int32),
            pltpu.VMEM((chunk, D), table.dtype),
            pltpu.VMEM((chunk, D), table.dtype),
            pltpu.SemaphoreType.DMA,
            pltpu.SemaphoreType.DMA,
        ],
    )
    def k(t_hbm, i_hbm, o_hbm, i0, i1, r0, r1, sg, ss):
        cidx = lax.axis_index("c")
        sidx = lax.axis_index("s")
        my_n = jnp.where(cidx == 0, c0, c1)
        base = jnp.where(cidx == 0, sidx * c0,
                         16 * c0 + sidx * c1) * chunk
        nb2 = my_n // 2

        pltpu.sync_copy(i_hbm.at[pl.ds(base, chunk)], i0)
        pltpu.async_copy(t_hbm.at[i0], r0, sg)

        @pl.loop(0, nb2)
        def _(jj):
            ea = base + 2 * jj * chunk
            pltpu.sync_copy(i_hbm.at[pl.ds(ea + chunk, chunk)], i1)

            @pl.when(jj > 0)
            def _():
                pltpu.make_async_copy(r1, o_hbm.at[pl.ds(ea - chunk, chunk)],
                                      ss).wait()

            pltpu.async_copy(t_hbm.at[i1], r1, sg)
            pltpu.make_async_copy(t_hbm.at[i0], r0, sg).wait()
            pltpu.async_copy(r0, o_hbm.at[pl.ds(ea, chunk)], ss)

            @pl.when(jj < nb2 - 1)
            def _():
                pltpu.sync_copy(i_hbm.at[pl.ds(ea + 2 * chunk, chunk)], i0)

            pltpu.make_async_copy(t_hbm.at[i1], r1, sg).wait()
            pltpu.async_copy(r1, o_hbm.at[pl.ds(ea + chunk, chunk)], ss)

            @pl.when(jj < nb2 - 1)
            def _():
                pltpu.make_async_copy(r0, o_hbm.at[pl.ds(ea, chunk)],
                                      ss).wait()
                pltpu.async_copy(t_hbm.at[i0], r0, sg)

        end = base + (my_n - 2) * chunk
        pltpu.make_async_copy(r0, o_hbm.at[pl.ds(end, chunk)], ss).wait()
        pltpu.make_async_copy(r1, o_hbm.at[pl.ds(end + chunk, chunk)],
                              ss).wait()

    return k(table, idx)


def _prep(xz8, xyz8, W1p, W1at):
    def body(xz_r, xyz_r, w_r, wa_r, g_r, c_r):
        g_r[...] = jnp.dot(xz_r[...], w_r[...],
                           preferred_element_type=jnp.float32)
        c_r[...] = jnp.dot(xyz_r[...], wa_r[...],
                           preferred_element_type=jnp.float32)

    return pl.pallas_call(
        body,
        out_shape=(jax.ShapeDtypeStruct((N_PAD, 16), jnp.float32),
                   jax.ShapeDtypeStruct((N_PAD, DIM), jnp.float32)),
    )(xz8, xyz8, W1p, W1at)


P_B = 1024


def _edge(G2, ct, b1wt, b1bt, BD2, b2wt, b2bt, W3, nbw, nbb, L, lw, lb):
    grid = N_PAD // P_B

    def body(G_r, c_r, b1w_r, b1b_r, W2_r, b2w_r, b2b_r, W3_r, nw_r, nb_r,
             L_r, lw_r, lb_r, xf_r, y_r):
        bf = jnp.bfloat16
        h1 = jax.nn.gelu((G_r[...] - c_r[...]) * b1w_r[...] + b1b_r[...])
        h2 = jax.nn.gelu(
            jnp.dot(h1.astype(bf), W2_r[...].astype(bf),
                    preferred_element_type=jnp.float32)
            * b2w_r[...] + b2b_r[...])
        h2b = h2.astype(bf)
        W3b = W3_r[...].astype(bf)
        m = jnp.dot(h2b[:, 0:32], W3b, preferred_element_type=jnp.float32)
        for kk in range(1, K):
            m = jnp.maximum(
                m, jnp.dot(h2b[:, kk * 32:(kk + 1) * 32], W3b,
                           preferred_element_type=jnp.float32))
        xf = m * nw_r[...] + nb_r[...]
        xf_r[...] = xf
        y = (jnp.dot(xf.astype(bf), L_r[...].astype(bf),
                     preferred_element_type=jnp.float32)
             * lw_r[...] + lb_r[...])
        y16 = y.astype(jnp.bfloat16)
        au = lax.bitcast_convert_type(y16[:, :128],
                                      jnp.uint16).astype(jnp.uint32)
        bu = lax.bitcast_convert_type(y16[:, 128:],
                                      jnp.uint16).astype(jnp.uint32)
        y_r[...] = lax.bitcast_convert_type(au | (bu << 16), jnp.int32)

    return pl.pallas_call(
        body,
        grid=(grid,),
        in_specs=[
            pl.BlockSpec((P_B, DIM), lambda i: (i, 0)),
            pl.BlockSpec((P_B, DIM), lambda i: (i, 0)),
            pl.BlockSpec((1, DIM), lambda i: (0, 0)),
            pl.BlockSpec((1, DIM), lambda i: (0, 0)),
            pl.BlockSpec((DIM, 2 * DIM), lambda i: (0, 0)),
            pl.BlockSpec((1, 2 * DIM), lambda i: (0, 0)),
            pl.BlockSpec((1, 2 * DIM), lambda i: (0, 0)),
            pl.BlockSpec((32, DIM), lambda i: (0, 0)),
            pl.BlockSpec((1, DIM), lambda i: (0, 0)),
            pl.BlockSpec((1, DIM), lambda i: (0, 0)),
            pl.BlockSpec((DIM, DIM), lambda i: (0, 0)),
            pl.BlockSpec((1, DIM), lambda i: (0, 0)),
            pl.BlockSpec((1, DIM), lambda i: (0, 0)),
        ],
        out_specs=(pl.BlockSpec((P_B, DIM), lambda i: (i, 0)),
                   pl.BlockSpec((P_B, 128), lambda i: (i, 0))),
        out_shape=(jax.ShapeDtypeStruct((N_PAD, DIM), jnp.float32),
                   jax.ShapeDtypeStruct((N_PAD, 128), jnp.int32)),
    )(G2, ct, b1wt, b1bt, BD2, b2wt, b2bt, W3, nbw, nbb, L, lw, lb)


C_E = 1024
NH = N_PAD // 2


def _final(xf, ym, bw, bb, M1, M2, Win, Wa, Wout, pw, pb, P, off, cin):
    grid = NH // C_E

    def body(xf_r, ym_r, bw_r, bb_r, M1_r, M2_r, Wi_r, Wa_r, Wo_r, pw_r,
             pb_r, P_r, cin_r, o_r, co_r, carry_r):
        i = pl.program_id(0)

        @pl.when(i == 0)
        def _():
            carry_r[...] = cin_r[0:1, :]

        u32 = lax.bitcast_convert_type(ym_r[...], jnp.uint32)
        lo = lax.bitcast_convert_type((u32 & 0xFFFF).astype(jnp.uint16),
                                      jnp.bfloat16)
        hi = lax.bitcast_convert_type((u32 >> 16).astype(jnp.uint16),
                                      jnp.bfloat16)
        mlo = lo[0]
        mhi = hi[0]
        for kk in range(1, K):
            mlo = jnp.maximum(mlo, lo[kk])
            mhi = jnp.maximum(mhi, hi[kk])
        ym = jnp.concatenate([mlo, mhi], axis=1).astype(jnp.float32)

        bf = jnp.bfloat16
        xf2 = xf_r[...] + ym
        t = xf2 * bw_r[...] + bb_r[...]
        y2 = jnp.dot(
            jax.nn.gelu(jnp.dot(t.astype(bf), M1_r[...].astype(bf),
                                preferred_element_type=jnp.float32)
                        ).astype(bf),
            M2_r[...].astype(bf), preferred_element_type=jnp.float32)
        xf3 = xf2 + y2
        xf3b = xf3.astype(bf)
        z = jnp.dot(xf3b, Wi_r[...].astype(bf),
                    preferred_element_type=jnp.float32)
        u = z[:, :DIM]
        gg = z[:, DIM:]
        a = jax.nn.sigmoid(jnp.dot(xf3b, Wa_r[...].astype(bf),
                                   preferred_element_type=jnp.float32))
        h = u
        A = a
        s = 1
        while s < C_E:
            h = h + A * jnp.concatenate(
                [jnp.zeros((s, DIM), jnp.float32), h[:C_E - s]], axis=0)
            A = A * jnp.concatenate(
                [jnp.ones((s, DIM), jnp.float32), A[:C_E - s]], axis=0)
            s *= 2
        hh = h + A * carry_r[...]
        carry_r[...] = hh[C_E - 1:C_E, :]
        co_r[...] = jnp.broadcast_to(hh[C_E - 1:C_E, :], (8, DIM))
        xf4 = xf3 + jnp.dot((jax.nn.silu(gg) * hh).astype(bf),
                            Wo_r[...].astype(bf),
                            preferred_element_type=jnp.float32)
        o_r[...] = jnp.dot((xf4 * pw_r[...] + pb_r[...]).astype(bf),
                           P_r[...].astype(bf),
                           preferred_element_type=jnp.float32)

    return pl.pallas_call(
        body,
        grid=(grid,),
        in_specs=[
            pl.BlockSpec((C_E, DIM), lambda i, off=off: (i + off, 0)),
            pl.BlockSpec((K, C_E, 128), lambda i: (0, i, 0)),
            pl.BlockSpec((1, DIM), lambda i: (0, 0)),
            pl.BlockSpec((1, DIM), lambda i: (0, 0)),
            pl.BlockSpec((DIM, 2 * DIM), lambda i: (0, 0)),
            pl.BlockSpec((2 * DIM, DIM), lambda i: (0, 0)),
            pl.BlockSpec((DIM, 2 * DIM), lambda i: (0, 0)),
            pl.BlockSpec((DIM, DIM), lambda i: (0, 0)),
            pl.BlockSpec((DIM, DIM), lambda i: (0, 0)),
            pl.BlockSpec((1, DIM), lambda i: (0, 0)),
            pl.BlockSpec((1, DIM), lambda i: (0, 0)),
            pl.BlockSpec((DIM, DIM), lambda i: (0, 0)),
            pl.BlockSpec((8, DIM), lambda i: (0, 0)),
        ],
        out_specs=(pl.BlockSpec((C_E, DIM), lambda i: (i, 0)),
                   pl.BlockSpec((8, DIM), lambda i: (0, 0))),
        out_shape=(jax.ShapeDtypeStruct((NH, DIM), jnp.float32),
                   jax.ShapeDtypeStruct((8, DIM), jnp.float32)),
        scratch_shapes=[pltpu.VMEM((1, DIM), jnp.float32)],
    )(xf, ym, bw, bb, M1, M2, Win, Wa, Wout, pw, pb, P, cin)


def kernel(x, xyz, knn, pts, W1, bn1w, bn1b, W2, bn2w, bn2b, W3, nbw, nbb,
           L, lw, lb, bw, bb, M1, M2, Win, Wa, Wout, pw, pb, P):
    f32 = jnp.float32
    n = x.shape[0]
    padn = N_PAD - n
    xp = jnp.pad(x.astype(f32), ((0, padn), (0, 0)))
    xyzp = jnp.pad(xyz.astype(f32), ((0, padn), (0, 0)))
    knnp = jnp.pad(knn.astype(jnp.int32), ((0, padn), (0, 0)))
    idxP = knnp.reshape(-1)
    knnT = knnp.T
    idxKa = knnT[:, :NH].reshape(-1)
    idxKb = knnT[:, NH:].reshape(-1)

    z1 = jnp.zeros((N_PAD, 1), f32)
    xz8 = jnp.concatenate([xyzp, xp, z1], axis=1)
    xyz8 = jnp.concatenate([xyzp, jnp.zeros((N_PAD, 5), f32)], axis=1)
    W1p = jnp.concatenate([W1, jnp.zeros((1, 16), f32)], axis=0)
    W1at = jnp.tile(
        jnp.concatenate([W1[:3], jnp.zeros((5, 16), f32)], axis=0), (1, K))
    BD2 = jnp.kron(jnp.eye(K, dtype=f32), W2)

    r1 = lambda v: v.reshape(1, -1)
    rt = lambda v: jnp.tile(v, K).reshape(1, -1)

    g, ct = _prep(xz8, xyz8, W1p, W1at)
    G = _sc_gather(g, idxP, 1280, tc_tiling=False)
    G2 = G.reshape(N_PAD, DIM)
    xf, y = _edge(G2, ct, rt(bn1w), rt(bn1b), BD2, rt(bn2w), rt(bn2b), W3,
                  r1(nbw), r1(nbb), L, r1(lw), r1(lb))
    Ya = _sc_gather(y, idxKa, 320)
    Yb = _sc_gather(y, idxKb, 320)
    Y3a = Ya.reshape(K, NH, 128)
    Y3b = Yb.reshape(K, NH, 128)
    cz = jnp.zeros((8, DIM), f32)
    o1, cr = _final(xf, Y3a, r1(bw), r1(bb), M1, M2, Win, Wa, Wout, r1(pw),
                    r1(pb), P, 0, cz)
    o2, _ = _final(xf, Y3b, r1(bw), r1(bb), M1, M2, Win, Wa, Wout, r1(pw),
                   r1(pb), P, NH // C_E, cr)
    out = jnp.concatenate([o1, o2], axis=0)
    return out[:n]


# speedup vs baseline: 1.0059x; 1.0059x over previous
import functools

import jax
import jax.numpy as jnp
from jax import lax
from jax.experimental import pallas as pl
from jax.experimental.pallas import tpu as pltpu
from jax.experimental.pallas import tpu_sc as plsc

N_PAD = 10240
K = 16
DIM = 256
NW = 32
E_TOT = N_PAD * K


def _sc_gather(table, idx, chunk, tc_tiling=True, c0=None):
    V, D = table.shape
    E = idx.shape[0]
    n_tot = E // chunk
    per_sub = n_tot // 16
    if c0 is None:
        c0 = per_sub // 2
    c1 = per_sub - c0
    assert c0 % 2 == 0 and c1 % 2 == 0 and c0 > 0 and c1 > 0
    mesh = plsc.VectorSubcoreMesh(core_axis_name="c", subcore_axis_name="s")

    @functools.partial(
        pl.kernel,
        mesh=mesh,
        compiler_params=pltpu.CompilerParams(use_tc_tiling_on_sc=tc_tiling),
        out_type=jax.ShapeDtypeStruct((E, D), table.dtype),
        scratch_types=[
            pltpu.VMEM((chunk,), jnp.int32),
            pltpu.VMEM((chunk,), jnp.int32),
            pltpu.VMEM((chunk, D), table.dtype),
            pltpu.VMEM((chunk, D), table.dtype),
            pltpu.SemaphoreType.DMA,
            pltpu.SemaphoreType.DMA,
        ],
    )
    def k(t_hbm, i_hbm, o_hbm, i0, i1, r0, r1, sg, ss):
        cidx = lax.axis_index("c")
        sidx = lax.axis_index("s")
        my_n = jnp.where(cidx == 0, c0, c1)
        base = jnp.where(cidx == 0, sidx * c0,
                         16 * c0 + sidx * c1) * chunk
        nb2 = my_n // 2

        pltpu.sync_copy(i_hbm.at[pl.ds(base, chunk)], i0)
        pltpu.async_copy(t_hbm.at[i0], r0, sg)

        @pl.loop(0, nb2)
        def _(jj):
            ea = base + 2 * jj * chunk
            pltpu.sync_copy(i_hbm.at[pl.ds(ea + chunk, chunk)], i1)

            @pl.when(jj > 0)
            def _():
                pltpu.make_async_copy(r1, o_hbm.at[pl.ds(ea - chunk, chunk)],
                                      ss).wait()

            pltpu.async_copy(t_hbm.at[i1], r1, sg)
            pltpu.make_async_copy(t_hbm.at[i0], r0, sg).wait()
            pltpu.async_copy(r0, o_hbm.at[pl.ds(ea, chunk)], ss)

            @pl.when(jj < nb2 - 1)
            def _():
                pltpu.sync_copy(i_hbm.at[pl.ds(ea + 2 * chunk, chunk)], i0)

            pltpu.make_async_copy(t_hbm.at[i1], r1, sg).wait()
            pltpu.async_copy(r1, o_hbm.at[pl.ds(ea + chunk, chunk)], ss)

            @pl.when(jj < nb2 - 1)
            def _():
                pltpu.make_async_copy(r0, o_hbm.at[pl.ds(ea, chunk)],
                                      ss).wait()
                pltpu.async_copy(t_hbm.at[i0], r0, sg)

        end = base + (my_n - 2) * chunk
        pltpu.make_async_copy(r0, o_hbm.at[pl.ds(end, chunk)], ss).wait()
        pltpu.make_async_copy(r1, o_hbm.at[pl.ds(end + chunk, chunk)],
                              ss).wait()

    return k(table, idx)


def _prep(xz8, xyz8, W1p, W1at):
    def body(xz_r, xyz_r, w_r, wa_r, g_r, c_r):
        g_r[...] = jnp.dot(xz_r[...], w_r[...],
                           preferred_element_type=jnp.float32)
        c_r[...] = jnp.dot(xyz_r[...], wa_r[...],
                           preferred_element_type=jnp.float32)

    return pl.pallas_call(
        body,
        out_shape=(jax.ShapeDtypeStruct((N_PAD, 16), jnp.float32),
                   jax.ShapeDtypeStruct((N_PAD, DIM), jnp.float32)),
    )(xz8, xyz8, W1p, W1at)


P_B = 1024


def _edge(G2, ct, b1wt, b1bt, BD2, b2wt, b2bt, W3, nbw, nbb, L, lw, lb):
    grid = N_PAD // P_B

    def body(G_r, c_r, b1w_r, b1b_r, W2_r, b2w_r, b2b_r, W3_r, nw_r, nb_r,
             L_r, lw_r, lb_r, xf_r, y_r):
        bf = jnp.bfloat16
        h1 = jax.nn.gelu((G_r[...] - c_r[...]) * b1w_r[...] + b1b_r[...])
        h2 = jax.nn.gelu(
            jnp.dot(h1.astype(bf), W2_r[...].astype(bf),
                    preferred_element_type=jnp.float32)
            * b2w_r[...] + b2b_r[...])
        h2b = h2.astype(bf)
        W3b = W3_r[...].astype(bf)
        m = jnp.dot(h2b[:, 0:32], W3b, preferred_element_type=jnp.float32)
        for kk in range(1, K):
            m = jnp.maximum(
                m, jnp.dot(h2b[:, kk * 32:(kk + 1) * 32], W3b,
                           preferred_element_type=jnp.float32))
        xf = m * nw_r[...] + nb_r[...]
        xf_r[...] = xf
        y = (jnp.dot(xf.astype(bf), L_r[...].astype(bf),
                     preferred_element_type=jnp.float32)
             * lw_r[...] + lb_r[...])
        y16 = y.astype(jnp.bfloat16)
        au = lax.bitcast_convert_type(y16[:, :128],
                                      jnp.uint16).astype(jnp.uint32)
        bu = lax.bitcast_convert_type(y16[:, 128:],
                                      jnp.uint16).astype(jnp.uint32)
        y_r[...] = lax.bitcast_convert_type(au | (bu << 16), jnp.int32)

    return pl.pallas_call(
        body,
        grid=(grid,),
        in_specs=[
            pl.BlockSpec((P_B, DIM), lambda i: (i, 0)),
            pl.BlockSpec((P_B, DIM), lambda i: (i, 0)),
            pl.BlockSpec((1, DIM), lambda i: (0, 0)),
            pl.BlockSpec((1, DIM), lambda i: (0, 0)),
            pl.BlockSpec((DIM, 2 * DIM), lambda i: (0, 0)),
            pl.BlockSpec((1, 2 * DIM), lambda i: (0, 0)),
            pl.BlockSpec((1, 2 * DIM), lambda i: (0, 0)),
            pl.BlockSpec((32, DIM), lambda i: (0, 0)),
            pl.BlockSpec((1, DIM), lambda i: (0, 0)),
            pl.BlockSpec((1, DIM), lambda i: (0, 0)),
            pl.BlockSpec((DIM, DIM), lambda i: (0, 0)),
            pl.BlockSpec((1, DIM), lambda i: (0, 0)),
            pl.BlockSpec((1, DIM), lambda i: (0, 0)),
        ],
        out_specs=(pl.BlockSpec((P_B, DIM), lambda i: (i, 0)),
                   pl.BlockSpec((P_B, 128), lambda i: (i, 0))),
        out_shape=(jax.ShapeDtypeStruct((N_PAD, DIM), jnp.float32),
                   jax.ShapeDtypeStruct((N_PAD, 128), jnp.int32)),
    )(G2, ct, b1wt, b1bt, BD2, b2wt, b2bt, W3, nbw, nbb, L, lw, lb)


C_E = 1024
NH = N_PAD // 2


def _final(xf, ym, bw, bb, M1, M2, Win, Wa, Wout, pw, pb, P, off, cin):
    grid = NH // C_E

    def body(xf_r, ym_r, bw_r, bb_r, M1_r, M2_r, Wi_r, Wa_r, Wo_r, pw_r,
             pb_r, P_r, cin_r, o_r, co_r, carry_r):
        i = pl.program_id(0)

        @pl.when(i == 0)
        def _():
            carry_r[...] = cin_r[0:1, :]

        u32 = lax.bitcast_convert_type(ym_r[...], jnp.uint32)
        lo = lax.bitcast_convert_type((u32 & 0xFFFF).astype(jnp.uint16),
                                      jnp.bfloat16)
        hi = lax.bitcast_convert_type((u32 >> 16).astype(jnp.uint16),
                                      jnp.bfloat16)
        mlo = lo[0]
        mhi = hi[0]
        for kk in range(1, K):
            mlo = jnp.maximum(mlo, lo[kk])
            mhi = jnp.maximum(mhi, hi[kk])
        ym = jnp.concatenate([mlo, mhi], axis=1).astype(jnp.float32)

        bf = jnp.bfloat16
        xf2 = xf_r[...] + ym
        t = xf2 * bw_r[...] + bb_r[...]
        y2 = jnp.dot(
            jax.nn.gelu(jnp.dot(t.astype(bf), M1_r[...].astype(bf),
                                preferred_element_type=jnp.float32)
                        ).astype(bf),
            M2_r[...].astype(bf), preferred_element_type=jnp.float32)
        xf3 = xf2 + y2
        xf3b = xf3.astype(bf)
        z = jnp.dot(xf3b, Wi_r[...].astype(bf),
                    preferred_element_type=jnp.float32)
        u = z[:, :DIM]
        gg = z[:, DIM:]
        a = jax.nn.sigmoid(jnp.dot(xf3b, Wa_r[...].astype(bf),
                                   preferred_element_type=jnp.float32))
        h = u
        A = a
        s = 1
        while s < C_E:
            h = h + A * jnp.concatenate(
                [jnp.zeros((s, DIM), jnp.float32), h[:C_E - s]], axis=0)
            A = A * jnp.concatenate(
                [jnp.ones((s, DIM), jnp.float32), A[:C_E - s]], axis=0)
            s *= 2
        hh = h + A * carry_r[...]
        carry_r[...] = hh[C_E - 1:C_E, :]
        co_r[...] = jnp.broadcast_to(hh[C_E - 1:C_E, :], (8, DIM))
        xf4 = xf3 + jnp.dot((jax.nn.silu(gg) * hh).astype(bf),
                            Wo_r[...].astype(bf),
                            preferred_element_type=jnp.float32)
        o_r[...] = jnp.dot((xf4 * pw_r[...] + pb_r[...]).astype(bf),
                           P_r[...].astype(bf),
                           preferred_element_type=jnp.float32)

    return pl.pallas_call(
        body,
        grid=(grid,),
        in_specs=[
            pl.BlockSpec((C_E, DIM), lambda i, off=off: (i + off, 0)),
            pl.BlockSpec((K, C_E, 128), lambda i: (0, i, 0)),
            pl.BlockSpec((1, DIM), lambda i: (0, 0)),
            pl.BlockSpec((1, DIM), lambda i: (0, 0)),
            pl.BlockSpec((DIM, 2 * DIM), lambda i: (0, 0)),
            pl.BlockSpec((2 * DIM, DIM), lambda i: (0, 0)),
            pl.BlockSpec((DIM, 2 * DIM), lambda i: (0, 0)),
            pl.BlockSpec((DIM, DIM), lambda i: (0, 0)),
            pl.BlockSpec((DIM, DIM), lambda i: (0, 0)),
            pl.BlockSpec((1, DIM), lambda i: (0, 0)),
            pl.BlockSpec((1, DIM), lambda i: (0, 0)),
            pl.BlockSpec((DIM, DIM), lambda i: (0, 0)),
            pl.BlockSpec((8, DIM), lambda i: (0, 0)),
        ],
        out_specs=(pl.BlockSpec((C_E, DIM), lambda i: (i, 0)),
                   pl.BlockSpec((8, DIM), lambda i: (0, 0))),
        out_shape=(jax.ShapeDtypeStruct((NH, DIM), jnp.float32),
                   jax.ShapeDtypeStruct((8, DIM), jnp.float32)),
        scratch_shapes=[pltpu.VMEM((1, DIM), jnp.float32)],
    )(xf, ym, bw, bb, M1, M2, Win, Wa, Wout, pw, pb, P, cin)


def kernel(x, xyz, knn, pts, W1, bn1w, bn1b, W2, bn2w, bn2b, W3, nbw, nbb,
           L, lw, lb, bw, bb, M1, M2, Win, Wa, Wout, pw, pb, P):
    f32 = jnp.float32
    n = x.shape[0]
    padn = N_PAD - n
    xp = jnp.pad(x.astype(f32), ((0, padn), (0, 0)))
    xyzp = jnp.pad(xyz.astype(f32), ((0, padn), (0, 0)))
    knnp = jnp.pad(knn.astype(jnp.int32), ((0, padn), (0, 0)))
    idxP = knnp.reshape(-1)
    knnT = knnp.T
    idxKa = knnT[:, :NH].reshape(-1)
    idxKb = knnT[:, NH:].reshape(-1)

    z1 = jnp.zeros((N_PAD, 1), f32)
    xz8 = jnp.concatenate([xyzp, xp, z1], axis=1)
    xyz8 = jnp.concatenate([xyzp, jnp.zeros((N_PAD, 5), f32)], axis=1)
    W1p = jnp.concatenate([W1, jnp.zeros((1, 16), f32)], axis=0)
    W1at = jnp.tile(
        jnp.concatenate([W1[:3], jnp.zeros((5, 16), f32)], axis=0), (1, K))
    BD2 = jnp.kron(jnp.eye(K, dtype=f32), W2)

    r1 = lambda v: v.reshape(1, -1)
    rt = lambda v: jnp.tile(v, K).reshape(1, -1)

    g, ct = _prep(xz8, xyz8, W1p, W1at)
    G = _sc_gather(g, idxP, 1280, tc_tiling=False)
    G2 = G.reshape(N_PAD, DIM)
    xf, y = _edge(G2, ct, rt(bn1w), rt(bn1b), BD2, rt(bn2w), rt(bn2b), W3,
                  r1(nbw), r1(nbb), L, r1(lw), r1(lb))
    Ya = _sc_gather(y, idxKa, 128)
    Yb = _sc_gather(y, idxKb, 128)
    Y3a = Ya.reshape(K, NH, 128)
    Y3b = Yb.reshape(K, NH, 128)
    cz = jnp.zeros((8, DIM), f32)
    o1, cr = _final(xf, Y3a, r1(bw), r1(bb), M1, M2, Win, Wa, Wout, r1(pw),
                    r1(pb), P, 0, cz)
    o2, _ = _final(xf, Y3b, r1(bw), r1(bb), M1, M2, Win, Wa, Wout, r1(pw),
                   r1(pb), P, NH // C_E, cr)
    out = jnp.concatenate([o1, o2], axis=0)
    return out[:n]
